# baseline (speedup 1.0000x reference)
import functools
import math

import jax
import jax.numpy as jnp
from jax import lax
from jax.experimental import pallas as pl
from jax.experimental.pallas import tpu as pltpu
from jax.experimental.pallas import tpu_sc as plsc

D_MODEL = 64
SCALE = math.sqrt(D_MODEL)
LANES = 16

_NC = 2
_NS = 16
_NW = _NC * _NS

_CHUNK_ROWS = 4
_IDX_MINOR = 128
_CHUNK = _CHUNK_ROWS * _IDX_MINOR
_SUB = 32
_NSUB = _IDX_MINOR // _SUB


def _emb_kernel(n_rows_total, lut_hbm, idx_hbm, out_hbm,
                idx0, idx1, rows0, rows1, sem_g, sem_s):
    wid = lax.axis_index("s") * _NC + lax.axis_index("c")
    rows_per_w = n_rows_total // _NW
    n_chunks = rows_per_w // _CHUNK_ROWS
    base_row = wid * rows_per_w
    idx_bufs = (idx0, idx1)
    row_bufs = (rows0, rows1)

    def stage_and_fire(ch, b):
        row0 = base_row + ch * _CHUNK_ROWS
        pltpu.sync_copy(idx_hbm.at[pl.ds(row0, _CHUNK_ROWS)], idx_bufs[b])
        for j in range(_CHUNK_ROWS):
            for o in range(_IDX_MINOR // LANES):
                iv = idx_bufs[b][j, pl.ds(o * LANES, LANES)]
                pltpu.async_copy(
                    lut_hbm.at[iv],
                    row_bufs[b].at[
                        pl.ds(j * _IDX_MINOR + o * LANES, LANES)],
                    sem_g,
                )

    def wait_gather(b):
        dummy = lax.iota(jnp.int32, 16)
        for k in range(_CHUNK // LANES):
            pltpu.make_async_copy(
                lut_hbm.at[dummy],
                row_bufs[b].at[pl.ds(k * LANES, LANES)],
                sem_g,
            ).wait()

    def scale(b):
        rv = row_bufs[b]

        def body(r, carry):
            for u in range(8):
                rr = r * 8 + u
                for c in range(D_MODEL // LANES):
                    sl = pl.ds(c * LANES, LANES)
                    rv[rr, sl] = rv[rr, sl] * SCALE
            return carry

        lax.fori_loop(0, _CHUNK // 8, body, 0)

    def fire_store(ch, b):
        row0 = base_row + ch * _CHUNK_ROWS
        pltpu.async_copy(
            row_bufs[b], out_hbm.at[pl.ds(row0 * _IDX_MINOR, _CHUNK)], sem_s)

    def wait_store(b):
        pltpu.make_async_copy(
            row_bufs[b], out_hbm.at[pl.ds(0, _CHUNK)], sem_s).wait()

    stage_and_fire(0, 0)

    def outer(g, carry):
        for b in (0, 1):
            ch = 2 * g + b
            nxt = ch + 1

            @pl.when(nxt < n_chunks)
            def _():
                @pl.when(ch >= 1)
                def _():
                    wait_store(1 - b)

                stage_and_fire(nxt, 1 - b)

            wait_gather(b)
            scale(b)
            fire_store(ch, b)
        return carry

    lax.fori_loop(0, n_chunks // 2, outer, 0)
    wait_store(0)
    wait_store(1)


def kernel(x, lut):
    b, t = x.shape
    n = b * t
    assert n % (_NW * _CHUNK) == 0
    n_rows_total = n // _IDX_MINOR
    idx2d = x.reshape(n_rows_total, _IDX_MINOR).astype(jnp.int32)

    mesh = plsc.VectorSubcoreMesh(core_axis_name="c", subcore_axis_name="s")
    run = pl.kernel(
        functools.partial(_emb_kernel, n_rows_total),
        out_type=jax.ShapeDtypeStruct((n, D_MODEL), jnp.float32),
        mesh=mesh,
        scratch_types=[
            pltpu.VMEM((_CHUNK_ROWS, _IDX_MINOR), jnp.int32),
            pltpu.VMEM((_CHUNK_ROWS, _IDX_MINOR), jnp.int32),
            pltpu.VMEM((_CHUNK, D_MODEL), jnp.float32),
            pltpu.VMEM((_CHUNK, D_MODEL), jnp.float32),
            pltpu.SemaphoreType.DMA,
            pltpu.SemaphoreType.DMA,
        ],
        compiler_params=pltpu.CompilerParams(use_tc_tiling_on_sc=False),
    )
    out = run(lut, idx2d)
    return out.reshape(b, t, D_MODEL)


# speedup vs baseline: 1.0002x; 1.0002x over previous
import functools
import math

import jax
import jax.numpy as jnp
from jax import lax
from jax.experimental import pallas as pl
from jax.experimental.pallas import tpu as pltpu
from jax.experimental.pallas import tpu_sc as plsc

D_MODEL = 64
SCALE = math.sqrt(D_MODEL)
LANES = 16

_NC = 2
_NS = 16
_NW = _NC * _NS

_CHUNK = 512
_NSEM = 8


def _emb_kernel(n_idx, lut_hbm, idx_hbm, out_hbm,
                idx0, idx1, rows0, rows1, sem_s, *sem_g):
    wid = lax.axis_index("s") * _NC + lax.axis_index("c")
    per_w = n_idx // _NW
    n_chunks = per_w // _CHUNK
    base = wid * per_w
    idx_bufs = (idx0, idx1)
    row_bufs = (rows0, rows1)
    n_g = _CHUNK // LANES

    def stage_and_fire(ch, b):
        i0 = pl.multiple_of(base + ch * _CHUNK, _CHUNK)
        pltpu.sync_copy(idx_hbm.at[pl.ds(i0, _CHUNK)], idx_bufs[b])
        for o in range(n_g):
            iv = idx_bufs[b][pl.ds(o * LANES, LANES)]
            pltpu.async_copy(
                lut_hbm.at[iv],
                row_bufs[b].at[pl.ds(o * LANES, LANES)],
                sem_g[o % _NSEM],
            )

    def wait_gather(b):
        dummy = lax.iota(jnp.int32, 16)
        for o in range(n_g):
            pltpu.make_async_copy(
                lut_hbm.at[dummy],
                row_bufs[b].at[pl.ds(o * LANES, LANES)],
                sem_g[o % _NSEM],
            ).wait()

    def scale(b):
        rv = row_bufs[b]

        def body(r, carry):
            for u in range(8):
                rr = r * 8 + u
                for c in range(D_MODEL // LANES):
                    sl = pl.ds(c * LANES, LANES)
                    rv[rr, sl] = rv[rr, sl] * SCALE
            return carry

        lax.fori_loop(0, _CHUNK // 8, body, 0)

    def fire_store(ch, b):
        i0 = pl.multiple_of(base + ch * _CHUNK, _CHUNK)
        pltpu.async_copy(row_bufs[b], out_hbm.at[pl.ds(i0, _CHUNK)], sem_s)

    def wait_store(b):
        pltpu.make_async_copy(
            row_bufs[b], out_hbm.at[pl.ds(0, _CHUNK)], sem_s).wait()

    stage_and_fire(0, 0)

    def outer(g, carry):
        for b in (0, 1):
            ch = 2 * g + b
            nxt = ch + 1

            @pl.when(nxt < n_chunks)
            def _():
                @pl.when(ch >= 1)
                def _():
                    wait_store(1 - b)

                stage_and_fire(nxt, 1 - b)

            wait_gather(b)
            scale(b)
            fire_store(ch, b)
        return carry

    lax.fori_loop(0, n_chunks // 2, outer, 0)
    wait_store(0)
    wait_store(1)


def kernel(x, lut):
    b, t = x.shape
    n = b * t
    assert n % (_NW * _CHUNK) == 0
    idx = x.reshape(n).astype(jnp.int32)

    mesh = plsc.VectorSubcoreMesh(core_axis_name="c", subcore_axis_name="s")
    run = pl.kernel(
        functools.partial(_emb_kernel, n),
        out_type=jax.ShapeDtypeStruct((n, D_MODEL), jnp.float32),
        mesh=mesh,
        scratch_types=[
            pltpu.VMEM((_CHUNK,), jnp.int32),
            pltpu.VMEM((_CHUNK,), jnp.int32),
            pltpu.VMEM((_CHUNK, D_MODEL), jnp.float32),
            pltpu.VMEM((_CHUNK, D_MODEL), jnp.float32),
            pltpu.SemaphoreType.DMA,
        ] + [pltpu.SemaphoreType.DMA] * _NSEM,
        compiler_params=pltpu.CompilerParams(use_tc_tiling_on_sc=False),
    )
    out = run(lut, idx)
    return out.reshape(b, t, D_MODEL)


# speedup vs baseline: 1.0092x; 1.0090x over previous
import functools
import math

import jax
import jax.numpy as jnp
from jax import lax
from jax.experimental import pallas as pl
from jax.experimental.pallas import tpu as pltpu
from jax.experimental.pallas import tpu_sc as plsc

D_MODEL = 64
SCALE = math.sqrt(D_MODEL)
LANES = 16

_NC = 2
_NS = 16
_NW = _NC * _NS

_CHUNK_ROWS = 4
_IDX_MINOR = 128
_CHUNK = _CHUNK_ROWS * _IDX_MINOR


def _emb_kernel(n_rows_total, lut_hbm, idx_hbm, out_hbm,
                idx0, idx1, rows0, rows1, sem_g, sem_s):
    wid = lax.axis_index("s") * _NC + lax.axis_index("c")
    rows_per_w = n_rows_total // _NW
    n_chunks = rows_per_w // _CHUNK_ROWS
    base_row = wid * rows_per_w
    idx_bufs = (idx0, idx1)
    row_bufs = (rows0, rows1)

    def stage_and_fire(ch, b):
        row0 = base_row + ch * _CHUNK_ROWS
        pltpu.sync_copy(idx_hbm.at[pl.ds(row0, _CHUNK_ROWS)], idx_bufs[b])
        for j in range(_CHUNK_ROWS):
            pltpu.async_copy(
                lut_hbm.at[idx_bufs[b].at[j]],
                row_bufs[b].at[pl.ds(j * _IDX_MINOR, _IDX_MINOR)],
                sem_g,
            )

    def wait_gather(b):
        for j in range(_CHUNK_ROWS):
            pltpu.make_async_copy(
                lut_hbm.at[idx_bufs[b].at[j]],
                row_bufs[b].at[pl.ds(j * _IDX_MINOR, _IDX_MINOR)],
                sem_g,
            ).wait()

    def scale(b):
        rv = row_bufs[b]

        def body(r, carry):
            for u in range(8):
                rr = r * 8 + u
                for c in range(D_MODEL // LANES):
                    sl = pl.ds(c * LANES, LANES)
                    rv[rr, sl] = rv[rr, sl] * SCALE
            return carry

        lax.fori_loop(0, _CHUNK // 8, body, 0)

    def fire_store(ch, b):
        row0 = base_row + ch * _CHUNK_ROWS
        pltpu.async_copy(
            row_bufs[b], out_hbm.at[pl.ds(row0 * _IDX_MINOR, _CHUNK)], sem_s)

    def wait_store(b):
        pltpu.make_async_copy(
            row_bufs[b], out_hbm.at[pl.ds(0, _CHUNK)], sem_s).wait()

    stage_and_fire(0, 0)

    def outer(g, carry):
        for b in (0, 1):
            ch = 2 * g + b
            nxt = ch + 1

            @pl.when(nxt < n_chunks)
            def _():
                @pl.when(ch >= 1)
                def _():
                    wait_store(1 - b)

                stage_and_fire(nxt, 1 - b)

            wait_gather(b)
            scale(b)
            fire_store(ch, b)
        return carry

    lax.fori_loop(0, n_chunks // 2, outer, 0)
    wait_store(0)
    wait_store(1)


def kernel(x, lut):
    b, t = x.shape
    n = b * t
    assert n % (_NW * _CHUNK) == 0
    n_rows_total = n // _IDX_MINOR
    idx2d = x.reshape(n_rows_total, _IDX_MINOR).astype(jnp.int32)

    mesh = plsc.VectorSubcoreMesh(core_axis_name="c", subcore_axis_name="s")
    run = pl.kernel(
        functools.partial(_emb_kernel, n_rows_total),
        out_type=jax.ShapeDtypeStruct((n, D_MODEL), jnp.float32),
        mesh=mesh,
        scratch_types=[
            pltpu.VMEM((_CHUNK_ROWS, _IDX_MINOR), jnp.int32),
            pltpu.VMEM((_CHUNK_ROWS, _IDX_MINOR), jnp.int32),
            pltpu.VMEM((_CHUNK, D_MODEL), jnp.float32),
            pltpu.VMEM((_CHUNK, D_MODEL), jnp.float32),
            pltpu.SemaphoreType.DMA,
            pltpu.SemaphoreType.DMA,
        ],
        compiler_params=pltpu.CompilerParams(use_tc_tiling_on_sc=False),
    )
    out = run(lut, idx2d)
    return out.reshape(b, t, D_MODEL)


# speedup vs baseline: 1.0184x; 1.0091x over previous
import functools
import math

import jax
import jax.numpy as jnp
from jax import lax
from jax.experimental import pallas as pl
from jax.experimental.pallas import tpu as pltpu
from jax.experimental.pallas import tpu_sc as plsc

D_MODEL = 64
SCALE = math.sqrt(D_MODEL)
LANES = 16

_NC = 2
_NS = 16
_NW = _NC * _NS

_CHUNK_ROWS = 2
_IDX_MINOR = 128
_CHUNK = _CHUNK_ROWS * _IDX_MINOR
_NBUF = 4


def _emb_kernel(n_rows_total, lut_hbm, idx_hbm, out_hbm,
                idx0, idx1, idx2, idx3, rows0, rows1, rows2, rows3,
                sem_g, sem_s):
    wid = lax.axis_index("s") * _NC + lax.axis_index("c")
    rows_per_w = n_rows_total // _NW
    n_chunks = rows_per_w // _CHUNK_ROWS
    base_row = wid * rows_per_w
    idx_bufs = (idx0, idx1, idx2, idx3)
    row_bufs = (rows0, rows1, rows2, rows3)

    def stage_and_fire(ch, b):
        row0 = base_row + ch * _CHUNK_ROWS
        pltpu.sync_copy(idx_hbm.at[pl.ds(row0, _CHUNK_ROWS)], idx_bufs[b])
        for j in range(_CHUNK_ROWS):
            pltpu.async_copy(
                lut_hbm.at[idx_bufs[b].at[j]],
                row_bufs[b].at[pl.ds(j * _IDX_MINOR, _IDX_MINOR)],
                sem_g,
            )

    def wait_gather(b):
        for j in range(_CHUNK_ROWS):
            pltpu.make_async_copy(
                lut_hbm.at[idx_bufs[b].at[j]],
                row_bufs[b].at[pl.ds(j * _IDX_MINOR, _IDX_MINOR)],
                sem_g,
            ).wait()

    def scale(b):
        rv = row_bufs[b]

        def body(r, carry):
            for u in range(8):
                rr = r * 8 + u
                for c in range(D_MODEL // LANES):
                    sl = pl.ds(c * LANES, LANES)
                    rv[rr, sl] = rv[rr, sl] * SCALE
            return carry

        lax.fori_loop(0, _CHUNK // 8, body, 0)

    def fire_store(ch, b):
        row0 = base_row + ch * _CHUNK_ROWS
        pltpu.async_copy(
            row_bufs[b], out_hbm.at[pl.ds(row0 * _IDX_MINOR, _CHUNK)], sem_s)

    def wait_store(b):
        pltpu.make_async_copy(
            row_bufs[b], out_hbm.at[pl.ds(0, _CHUNK)], sem_s).wait()

    stage_and_fire(0, 0)
    stage_and_fire(1, 1)

    def outer(g, carry):
        for b in range(_NBUF):
            ch = _NBUF * g + b
            nxt = ch + 2
            nb = (b + 2) % _NBUF

            @pl.when(nxt < n_chunks)
            def _():
                @pl.when(ch >= 2)
                def _():
                    wait_store(nb)

                stage_and_fire(nxt, nb)

            wait_gather(b)
            scale(b)
            fire_store(ch, b)
        return carry

    lax.fori_loop(0, n_chunks // _NBUF, outer, 0)
    for b in range(_NBUF):
        wait_store(b)


def kernel(x, lut):
    b, t = x.shape
    n = b * t
    assert n % (_NW * _CHUNK) == 0
    n_rows_total = n // _IDX_MINOR
    idx2d = x.reshape(n_rows_total, _IDX_MINOR).astype(jnp.int32)

    mesh = plsc.VectorSubcoreMesh(core_axis_name="c", subcore_axis_name="s")
    run = pl.kernel(
        functools.partial(_emb_kernel, n_rows_total),
        out_type=jax.ShapeDtypeStruct((n, D_MODEL), jnp.float32),
        mesh=mesh,
        scratch_types=(
            [pltpu.VMEM((_CHUNK_ROWS, _IDX_MINOR), jnp.int32)] * _NBUF
            + [pltpu.VMEM((_CHUNK, D_MODEL), jnp.float32)] * _NBUF
            + [pltpu.SemaphoreType.DMA, pltpu.SemaphoreType.DMA]
        ),
        compiler_params=pltpu.CompilerParams(use_tc_tiling_on_sc=False),
    )
    out = run(lut, idx2d)
    return out.reshape(b, t, D_MODEL)


# speedup vs baseline: 1.0259x; 1.0074x over previous
import functools
import math

import jax
import jax.numpy as jnp
from jax import lax
from jax.experimental import pallas as pl
from jax.experimental.pallas import tpu as pltpu
from jax.experimental.pallas import tpu_sc as plsc

D_MODEL = 64
SCALE = math.sqrt(D_MODEL)
LANES = 16

_NC = 2
_NS = 16
_NW = _NC * _NS

_CHUNK_ROWS = 2
_IDX_MINOR = 128
_CHUNK = _CHUNK_ROWS * _IDX_MINOR
_NBUF = 4


def _emb_kernel(n_rows_total, lut_hbm, idx_hbm, out_hbm,
                idx0, idx1, idx2, idx3, rows0, rows1, rows2, rows3,
                sem_g, sem_s, sem_i):
    wid = lax.axis_index("s") * _NC + lax.axis_index("c")
    rows_per_w = n_rows_total // _NW
    n_chunks = rows_per_w // _CHUNK_ROWS
    base_row = wid * rows_per_w
    idx_bufs = (idx0, idx1, idx2, idx3)
    row_bufs = (rows0, rows1, rows2, rows3)

    def stage_idx(ch, b):
        row0 = base_row + ch * _CHUNK_ROWS
        pltpu.async_copy(
            idx_hbm.at[pl.ds(row0, _CHUNK_ROWS)], idx_bufs[b], sem_i)

    def fire_gathers(b):
        pltpu.make_async_copy(
            idx_hbm.at[pl.ds(0, _CHUNK_ROWS)], idx_bufs[b], sem_i).wait()
        for j in range(_CHUNK_ROWS):
            pltpu.async_copy(
                lut_hbm.at[idx_bufs[b].at[j]],
                row_bufs[b].at[pl.ds(j * _IDX_MINOR, _IDX_MINOR)],
                sem_g,
            )

    def wait_gather(b):
        for j in range(_CHUNK_ROWS):
            pltpu.make_async_copy(
                lut_hbm.at[idx_bufs[b].at[j]],
                row_bufs[b].at[pl.ds(j * _IDX_MINOR, _IDX_MINOR)],
                sem_g,
            ).wait()

    def scale(b):
        rv = row_bufs[b]

        def body(r, carry):
            for u in range(8):
                rr = r * 8 + u
                for c in range(D_MODEL // LANES):
                    sl = pl.ds(c * LANES, LANES)
                    rv[rr, sl] = rv[rr, sl] * SCALE
            return carry

        lax.fori_loop(0, _CHUNK // 8, body, 0)

    def fire_store(ch, b):
        row0 = base_row + ch * _CHUNK_ROWS
        pltpu.async_copy(
            row_bufs[b], out_hbm.at[pl.ds(row0 * _IDX_MINOR, _CHUNK)], sem_s)

    def wait_store(b):
        pltpu.make_async_copy(
            row_bufs[b], out_hbm.at[pl.ds(0, _CHUNK)], sem_s).wait()

    stage_idx(0, 0)
    stage_idx(1, 1)
    stage_idx(2, 2)
    fire_gathers(0)
    fire_gathers(1)

    def outer(g, carry):
        for b in range(_NBUF):
            ch = _NBUF * g + b
            nxt = ch + 2
            nb = (b + 2) % _NBUF

            @pl.when(ch + 3 < n_chunks)
            def _():
                stage_idx(ch + 3, (b + 3) % _NBUF)

            @pl.when(nxt < n_chunks)
            def _():
                @pl.when(ch >= 2)
                def _():
                    wait_store(nb)

                fire_gathers(nb)

            wait_gather(b)
            scale(b)
            fire_store(ch, b)
        return carry

    lax.fori_loop(0, n_chunks // _NBUF, outer, 0)
    for b in range(_NBUF):
        wait_store(b)


def kernel(x, lut):
    b, t = x.shape
    n = b * t
    assert n % (_NW * _CHUNK) == 0
    n_rows_total = n // _IDX_MINOR
    idx2d = x.reshape(n_rows_total, _IDX_MINOR).astype(jnp.int32)

    mesh = plsc.VectorSubcoreMesh(core_axis_name="c", subcore_axis_name="s")
    run = pl.kernel(
        functools.partial(_emb_kernel, n_rows_total),
        out_type=jax.ShapeDtypeStruct((n, D_MODEL), jnp.float32),
        mesh=mesh,
        scratch_types=(
            [pltpu.VMEM((_CHUNK_ROWS, _IDX_MINOR), jnp.int32)] * _NBUF
            + [pltpu.VMEM((_CHUNK, D_MODEL), jnp.float32)] * _NBUF
            + [pltpu.SemaphoreType.DMA] * 3
        ),
        compiler_params=pltpu.CompilerParams(use_tc_tiling_on_sc=False),
    )
    out = run(lut, idx2d)
    return out.reshape(b, t, D_MODEL)
